# Initial kernel scaffold; baseline (speedup 1.0000x reference)
import jax
import jax.numpy as jnp
from jax.experimental import pallas as pl


def kernel(points, features):
    raise NotImplementedError("write your pallas kernel here")


# speedup vs baseline: 28.1101x; 28.1101x over previous
import functools

import jax
import jax.numpy as jnp
from jax import lax
from jax.experimental import pallas as pl
from jax.experimental.pallas import tpu as pltpu
from jax.experimental.pallas import tpu_sc as plsc

B = 8
N = 8192
S = 1024
C = 128
CHUNK = 128


def _row_argmax(d, lanes):
    m = jnp.max(d, axis=1, keepdims=True)
    return jnp.min(jnp.where(d == m, lanes, N), axis=1, keepdims=True)


def _fps_body(px_ref, py_ref, pz_ref, idx_ref, cx_ref, cy_ref, cz_ref, dist_ref):
    X = px_ref[...]
    Y = py_ref[...]
    Z = pz_ref[...]
    lanes = lax.broadcasted_iota(jnp.int32, (B, N), 1)
    lanechunk = lax.broadcasted_iota(jnp.int32, (B, CHUNK), 1)
    rowoff = lax.broadcasted_iota(jnp.int32, (B, 1), 0) * N

    n = jnp.float32(N)
    mx = jnp.sum(X, axis=1, keepdims=True) / n
    my = jnp.sum(Y, axis=1, keepdims=True) / n
    mz = jnp.sum(Z, axis=1, keepdims=True) / n
    d0 = (X - mx) ** 2 + (Y - my) ** 2 + (Z - mz) ** 2
    far = _row_argmax(d0, lanes)

    dist_ref[...] = jnp.full((B, N), 1e10, jnp.float32)

    def body(j, carry):
        far, ia, xa, ya, za = carry
        oh = (lanes == far).astype(jnp.float32)
        cx = jnp.sum(X * oh, axis=1, keepdims=True)
        cy = jnp.sum(Y * oh, axis=1, keepdims=True)
        cz = jnp.sum(Z * oh, axis=1, keepdims=True)
        sel = lanechunk == j
        ia = jnp.where(sel, far + rowoff, ia)
        xa = jnp.where(sel, cx, xa)
        ya = jnp.where(sel, cy, ya)
        za = jnp.where(sel, cz, za)
        d = (X - cx) ** 2 + (Y - cy) ** 2 + (Z - cz) ** 2
        nd = jnp.minimum(dist_ref[...], d)
        dist_ref[...] = nd
        far = _row_argmax(nd, lanes)
        return far, ia, xa, ya, za

    zf = jnp.zeros((B, CHUNK), jnp.float32)
    zi = jnp.zeros((B, CHUNK), jnp.int32)
    for c in range(S // CHUNK):
        far, ia, xa, ya, za = lax.fori_loop(0, CHUNK, body, (far, zi, zf, zf, zf))
        sl = slice(c * CHUNK, (c + 1) * CHUNK)
        idx_ref[:, sl] = ia
        cx_ref[:, sl] = xa
        cy_ref[:, sl] = ya
        cz_ref[:, sl] = za


_fps = pl.pallas_call(
    _fps_body,
    out_shape=[
        jax.ShapeDtypeStruct((B, S), jnp.int32),
        jax.ShapeDtypeStruct((B, S), jnp.float32),
        jax.ShapeDtypeStruct((B, S), jnp.float32),
        jax.ShapeDtypeStruct((B, S), jnp.float32),
    ],
    scratch_shapes=[pltpu.VMEM((B, N), jnp.float32)],
)


def _make_gather():
    info = plsc.get_sparse_core_info()
    nw = info.num_cores * info.num_subcores
    per = (B * S) // nw
    mesh = plsc.VectorSubcoreMesh(core_axis_name="c", subcore_axis_name="s")

    @functools.partial(
        pl.kernel,
        mesh=mesh,
        out_type=jax.ShapeDtypeStruct((B * S, C), jnp.float32),
        scratch_types=[
            pltpu.VMEM((per,), jnp.int32),
            pltpu.VMEM((per, C), jnp.float32),
            pltpu.SemaphoreType.DMA,
        ],
    )
    def gather_k(table_hbm, idx_hbm, out_hbm, idx_v, rows_v, sem):
        wid = lax.axis_index("s") * info.num_cores + lax.axis_index("c")
        base = wid * per
        pltpu.sync_copy(idx_hbm.at[pl.ds(base, per)], idx_v)
        pltpu.async_copy(table_hbm.at[idx_v], rows_v, sem).wait()
        pltpu.sync_copy(rows_v, out_hbm.at[pl.ds(base, per)])

    return gather_k


@jax.jit
def kernel(points, features):
    px = points[:, 0, :]
    py = points[:, 1, :]
    pz = points[:, 2, :]
    gidx, cxo, cyo, czo = _fps(px, py, pz)
    sampled_points = jnp.stack([cxo, cyo, czo], axis=-1)
    table = jnp.swapaxes(features, -1, -2).reshape(B * N, C)
    flat = _make_gather()(table, gidx.reshape(B * S))
    sampled_features = flat.reshape(B, S, C)
    return sampled_points, sampled_features


# speedup vs baseline: 35.7275x; 1.2710x over previous
import functools

import jax
import jax.numpy as jnp
from jax import lax
from jax.experimental import pallas as pl
from jax.experimental.pallas import tpu as pltpu
from jax.experimental.pallas import tpu_sc as plsc

B = 8
N = 8192
S = 1024
C = 128
CHUNK = 128


NBLK = N // 128


def _fps_body(px_ref, py_ref, pz_ref, idx_ref, cx_ref, cy_ref, cz_ref, dist_ref):
    lane = lax.broadcasted_iota(jnp.int32, (B, 128), 1)
    rowoff = lax.broadcasted_iota(jnp.int32, (B, 1), 0) * N

    def sweep(cx, cy, cz, init_store):
        accs = [None, None]
        for a in range(2):
            accs[a] = (
                jnp.full((B, 128), -1.0, jnp.float32),
                jnp.zeros((B, 128), jnp.int32),
                jnp.zeros((B, 128), jnp.float32),
                jnp.zeros((B, 128), jnp.float32),
                jnp.zeros((B, 128), jnp.float32),
            )
        for k in range(NBLK):
            sl = slice(k * 128, (k + 1) * 128)
            Xk = px_ref[:, sl]
            Yk = py_ref[:, sl]
            Zk = pz_ref[:, sl]
            d = (Xk - cx) ** 2 + (Yk - cy) ** 2 + (Zk - cz) ** 2
            if init_store:
                dist_ref[:, sl] = jnp.full((B, 128), 1e10, jnp.float32)
                nd = d
            else:
                nd = jnp.minimum(dist_ref[:, sl], d)
                dist_ref[:, sl] = nd
            a = k & 1
            av, ab, ax, ay, az = accs[a]
            m = av >= nd
            kb = jnp.full((B, 128), k, jnp.int32)
            accs[a] = (
                jnp.where(m, av, nd),
                jnp.where(m, ab, kb),
                jnp.where(m, ax, Xk),
                jnp.where(m, ay, Yk),
                jnp.where(m, az, Zk),
            )
        (av1, ab1, ax1, ay1, az1), (av2, ab2, ax2, ay2, az2) = accs
        m12 = (av1 > av2) | ((av1 == av2) & (ab1 < ab2))
        av = jnp.where(m12, av1, av2)
        ab = jnp.where(m12, ab1, ab2)
        ax = jnp.where(m12, ax1, ax2)
        ay = jnp.where(m12, ay1, ay2)
        az = jnp.where(m12, az1, az2)
        mrow = jnp.max(av, axis=1, keepdims=True)
        gc = jnp.where(av == mrow, ab * 128 + lane, N)
        g = jnp.min(gc, axis=1, keepdims=True)
        oh = gc == g
        zf = jnp.zeros((B, 128), jnp.float32)
        ncx = jnp.sum(jnp.where(oh, ax, zf), axis=1, keepdims=True)
        ncy = jnp.sum(jnp.where(oh, ay, zf), axis=1, keepdims=True)
        ncz = jnp.sum(jnp.where(oh, az, zf), axis=1, keepdims=True)
        return g, ncx, ncy, ncz

    X = px_ref[...]
    Y = py_ref[...]
    Z = pz_ref[...]
    n = jnp.float32(N)
    mx = jnp.sum(X, axis=1, keepdims=True) / n
    my = jnp.sum(Y, axis=1, keepdims=True) / n
    mz = jnp.sum(Z, axis=1, keepdims=True) / n
    far, cx, cy, cz = sweep(mx, my, mz, True)

    def body(j, carry):
        far, cx, cy, cz, ia, xa, ya, za = carry
        sel = lane == j
        ia = jnp.where(sel, far + rowoff, ia)
        xa = jnp.where(sel, cx, xa)
        ya = jnp.where(sel, cy, ya)
        za = jnp.where(sel, cz, za)
        far, cx, cy, cz = sweep(cx, cy, cz, False)
        return far, cx, cy, cz, ia, xa, ya, za

    zf = jnp.zeros((B, CHUNK), jnp.float32)
    zi = jnp.zeros((B, CHUNK), jnp.int32)
    for c in range(S // CHUNK):
        far, cx, cy, cz, ia, xa, ya, za = lax.fori_loop(
            0, CHUNK, body, (far, cx, cy, cz, zi, zf, zf, zf)
        )
        sl = slice(c * CHUNK, (c + 1) * CHUNK)
        idx_ref[:, sl] = ia
        cx_ref[:, sl] = xa
        cy_ref[:, sl] = ya
        cz_ref[:, sl] = za


_fps = pl.pallas_call(
    _fps_body,
    out_shape=[
        jax.ShapeDtypeStruct((B, S), jnp.int32),
        jax.ShapeDtypeStruct((B, S), jnp.float32),
        jax.ShapeDtypeStruct((B, S), jnp.float32),
        jax.ShapeDtypeStruct((B, S), jnp.float32),
    ],
    scratch_shapes=[pltpu.VMEM((B, N), jnp.float32)],
)


def _make_gather():
    info = plsc.get_sparse_core_info()
    nw = info.num_cores * info.num_subcores
    per = (B * S) // nw
    mesh = plsc.VectorSubcoreMesh(core_axis_name="c", subcore_axis_name="s")

    @functools.partial(
        pl.kernel,
        mesh=mesh,
        out_type=jax.ShapeDtypeStruct((B * S, C), jnp.float32),
        scratch_types=[
            pltpu.VMEM((per,), jnp.int32),
            pltpu.VMEM((per, C), jnp.float32),
            pltpu.SemaphoreType.DMA,
        ],
    )
    def gather_k(table_hbm, idx_hbm, out_hbm, idx_v, rows_v, sem):
        wid = lax.axis_index("s") * info.num_cores + lax.axis_index("c")
        base = wid * per
        pltpu.sync_copy(idx_hbm.at[pl.ds(base, per)], idx_v)
        pltpu.async_copy(table_hbm.at[idx_v], rows_v, sem).wait()
        pltpu.sync_copy(rows_v, out_hbm.at[pl.ds(base, per)])

    return gather_k


@jax.jit
def kernel(points, features):
    px = points[:, 0, :]
    py = points[:, 1, :]
    pz = points[:, 2, :]
    gidx, cxo, cyo, czo = _fps(px, py, pz)
    sampled_points = jnp.stack([cxo, cyo, czo], axis=-1)
    table = jnp.swapaxes(features, -1, -2).reshape(B * N, C)
    flat = _make_gather()(table, gidx.reshape(B * S))
    sampled_features = flat.reshape(B, S, C)
    return sampled_points, sampled_features
